# baseline (speedup 1.0000x reference)
import functools

import jax
import jax.numpy as jnp
from jax.experimental import pallas as pl
from jax.experimental.pallas import tpu as pltpu

_N = 4096
_TP = 256
_KA = 8


def _norm_split(sq):
    hi = jnp.floor(sq * (1.0 / 256.0)) * 256.0
    return hi, sq - hi


def _vox_t(ct):
    cn = jnp.where(jnp.isnan(ct), jnp.inf, ct)
    mn = jnp.min(cn, axis=1, keepdims=True)
    return ((ct - mn) / 0.1).astype(jnp.int32).astype(jnp.float32)


def _chamfer_body(p_ref, g_ref, o_ref, pa_ref, ga_ref):
    p = jnp.transpose(p_ref[0], (1, 0))
    g = jnp.transpose(g_ref[0], (1, 0))
    pv = _vox_t(p)
    gv = _vox_t(g)
    shift = jnp.floor(jnp.maximum(jnp.max(pv, axis=1, keepdims=True),
                                  jnp.max(gv, axis=1, keepdims=True)) * 0.5)
    pv = pv - shift
    gv = gv - shift

    one_row = jnp.ones((1, _N), jnp.float32)
    pa_ref[7:8, :] = jnp.zeros((1, _N), jnp.float32)
    ga_ref[7:8, :] = jnp.zeros((1, _N), jnp.float32)
    total = jnp.float32(0.0)
    for pt, gt in ((p, g), (pv, gv)):
        xxh, xxl = _norm_split(jnp.sum(pt * pt, axis=0, keepdims=True))
        yyh, yyl = _norm_split(jnp.sum(gt * gt, axis=0, keepdims=True))
        pa_ref[0:3, :] = pt
        pa_ref[3:4, :] = xxh
        pa_ref[4:5, :] = xxl
        pa_ref[5:6, :] = one_row
        pa_ref[6:7, :] = one_row
        ga_ref[0:3, :] = -2.0 * gt
        ga_ref[3:4, :] = one_row
        ga_ref[4:5, :] = one_row
        ga_ref[5:6, :] = yyh
        ga_ref[6:7, :] = yyl
        ga = ga_ref[...]

        def step(c, carry):
            cacc, s1 = carry
            pc = pa_ref[:, pl.ds(c * _TP, _TP)]
            d = jax.lax.dot_general(
                pc, ga, (((0,), (0,)), ((), ())),
                preferred_element_type=jnp.float32)
            s1 = s1 + jnp.sum(jnp.min(d, axis=1))
            cacc = jnp.minimum(cacc, jnp.min(d, axis=0, keepdims=True))
            return cacc, s1

        cacc0 = jnp.full((1, _N), jnp.inf, dtype=jnp.float32)
        cacc, s1 = jax.lax.fori_loop(
            0, _N // _TP, step, (cacc0, jnp.float32(0.0)), unroll=16)
        total = total + s1 + jnp.sum(cacc)
    o_ref[0, 0, :] = jnp.full((128,), total, dtype=jnp.float32)


@jax.jit
def kernel(preds, gts):
    sums = pl.pallas_call(
        _chamfer_body,
        grid=(4,),
        in_specs=[
            pl.BlockSpec((1, _N, 3), lambda b: (b, 0, 0)),
            pl.BlockSpec((1, _N, 3), lambda b: (b, 0, 0)),
        ],
        out_specs=pl.BlockSpec((1, 1, 128), lambda b: (b, 0, 0)),
        out_shape=jax.ShapeDtypeStruct((4, 1, 128), jnp.float32),
        scratch_shapes=[
            pltpu.VMEM((_KA, _N), jnp.float32),
            pltpu.VMEM((_KA, _N), jnp.float32),
        ],
    )(preds, gts)

    return jnp.sum(sums[:, 0, 0]) / jnp.float32(4 * _N)


# speedup vs baseline: 1.1179x; 1.1179x over previous
import functools

import jax
import jax.numpy as jnp
from jax.experimental import pallas as pl
from jax.experimental.pallas import tpu as pltpu

_N = 4096
_TP = 256
_KA = 8


def _norm_split(sq):
    hi = jnp.floor(sq * (1.0 / 256.0)) * 256.0
    return hi, sq - hi


def _vox_t(ct):
    cn = jnp.where(jnp.isnan(ct), jnp.inf, ct)
    mn = jnp.min(cn, axis=1, keepdims=True)
    return ((ct - mn) / 0.1).astype(jnp.int32).astype(jnp.float32)


def _chamfer_body(p_ref, g_ref, o_ref, pa_ref, ga_ref):
    one_row = jnp.ones((1, _N), jnp.float32)
    pa_ref[7:8, :] = jnp.zeros((1, _N), jnp.float32)
    ga_ref[7:8, :] = jnp.zeros((1, _N), jnp.float32)
    total = jnp.float32(0.0)
    pairs = []
    for b in range(4):
        p = p_ref[b]
        g = g_ref[b]
        pv = _vox_t(p)
        gv = _vox_t(g)
        shift = jnp.floor(jnp.maximum(jnp.max(pv, axis=1, keepdims=True),
                                      jnp.max(gv, axis=1, keepdims=True)) * 0.5)
        pairs.append((p, g))
        pairs.append((pv - shift, gv - shift))
    for pt, gt in pairs:
        xxh, xxl = _norm_split(jnp.sum(pt * pt, axis=0, keepdims=True))
        yyh, yyl = _norm_split(jnp.sum(gt * gt, axis=0, keepdims=True))
        pa_ref[0:3, :] = pt
        pa_ref[3:4, :] = xxh
        pa_ref[4:5, :] = xxl
        pa_ref[5:6, :] = one_row
        pa_ref[6:7, :] = one_row
        ga_ref[0:3, :] = -2.0 * gt
        ga_ref[3:4, :] = one_row
        ga_ref[4:5, :] = one_row
        ga_ref[5:6, :] = yyh
        ga_ref[6:7, :] = yyl
        ga = ga_ref[...]

        def step(c, carry):
            cacc, s1 = carry
            pc = pa_ref[:, pl.ds(c * _TP, _TP)]
            d = jax.lax.dot_general(
                pc, ga, (((0,), (0,)), ((), ())),
                preferred_element_type=jnp.float32)
            s1 = s1 + jnp.sum(jnp.min(d, axis=1))
            cacc = jnp.minimum(cacc, jnp.min(d, axis=0, keepdims=True))
            return cacc, s1

        cacc0 = jnp.full((1, _N), jnp.inf, dtype=jnp.float32)
        cacc, s1 = jax.lax.fori_loop(
            0, _N // _TP, step, (cacc0, jnp.float32(0.0)), unroll=16)
        total = total + s1 + jnp.sum(cacc)
    o_ref[...] = jnp.full((8, 128), total, dtype=jnp.float32)


@jax.jit
def kernel(preds, gts):
    p_t = preds.transpose(0, 2, 1)
    g_t = gts.transpose(0, 2, 1)

    sums = pl.pallas_call(
        _chamfer_body,
        out_shape=jax.ShapeDtypeStruct((8, 128), jnp.float32),
        scratch_shapes=[
            pltpu.VMEM((_KA, _N), jnp.float32),
            pltpu.VMEM((_KA, _N), jnp.float32),
        ],
    )(p_t, g_t)

    return sums[0, 0] / jnp.float32(4 * _N)


# speedup vs baseline: 1.1774x; 1.0532x over previous
import functools

import jax
import jax.numpy as jnp
from jax.experimental import pallas as pl
from jax.experimental.pallas import tpu as pltpu

_N = 4096
_TP = 256
_KA = 8


def _norm_split(sq):
    hi = jnp.floor(sq * (1.0 / 256.0)) * 256.0
    return hi, sq - hi


def _vox_t(ct):
    cn = jnp.where(jnp.isnan(ct), jnp.inf, ct)
    mn = jnp.min(cn, axis=1, keepdims=True)
    return ((ct - mn) / 0.1).astype(jnp.int32).astype(jnp.float32)


def _chamfer_body(p_ref, g_ref, o_ref, pa_ref, ga_ref):
    p = p_ref[0]
    g = g_ref[0]
    pv = _vox_t(p)
    gv = _vox_t(g)
    shift = jnp.floor(jnp.maximum(jnp.max(pv, axis=1, keepdims=True),
                                  jnp.max(gv, axis=1, keepdims=True)) * 0.5)
    pv = pv - shift
    gv = gv - shift

    one_row = jnp.ones((1, _N), jnp.float32)
    pa_ref[7:8, :] = jnp.zeros((1, _N), jnp.float32)
    ga_ref[7:8, :] = jnp.zeros((1, _N), jnp.float32)
    total = jnp.float32(0.0)
    for pt, gt in ((p, g), (pv, gv)):
        xxh, xxl = _norm_split(jnp.sum(pt * pt, axis=0, keepdims=True))
        yyh, yyl = _norm_split(jnp.sum(gt * gt, axis=0, keepdims=True))
        pa_ref[0:3, :] = pt
        pa_ref[3:4, :] = xxh
        pa_ref[4:5, :] = xxl
        pa_ref[5:6, :] = one_row
        pa_ref[6:7, :] = one_row
        ga_ref[0:3, :] = -2.0 * gt
        ga_ref[3:4, :] = one_row
        ga_ref[4:5, :] = one_row
        ga_ref[5:6, :] = yyh
        ga_ref[6:7, :] = yyl
        ga = ga_ref[...]

        def step(c, carry):
            cacc, s1 = carry
            pc = pa_ref[:, pl.ds(c * _TP, _TP)]
            d = jax.lax.dot_general(
                pc, ga, (((0,), (0,)), ((), ())),
                preferred_element_type=jnp.float32)
            s1 = s1 + jnp.sum(jnp.min(d, axis=1))
            cacc = jnp.minimum(cacc, jnp.min(d, axis=0, keepdims=True))
            return cacc, s1

        cacc0 = jnp.full((1, _N), jnp.inf, dtype=jnp.float32)
        cacc, s1 = jax.lax.fori_loop(
            0, _N // _TP, step, (cacc0, jnp.float32(0.0)), unroll=16)
        total = total + s1 + jnp.sum(cacc)
    b = pl.program_id(0)

    @pl.when(b == 0)
    def _():
        o_ref[0] = jnp.float32(0.0)

    o_ref[0] = o_ref[0] + total * jnp.float32(1.0 / (4 * _N))


@jax.jit
def kernel(preds, gts):
    p_t = preds.transpose(0, 2, 1)
    g_t = gts.transpose(0, 2, 1)

    sums = pl.pallas_call(
        _chamfer_body,
        grid=(4,),
        in_specs=[
            pl.BlockSpec((1, 3, _N), lambda b: (b, 0, 0)),
            pl.BlockSpec((1, 3, _N), lambda b: (b, 0, 0)),
        ],
        out_specs=pl.BlockSpec(memory_space=pltpu.SMEM),
        out_shape=jax.ShapeDtypeStruct((1,), jnp.float32),
        scratch_shapes=[
            pltpu.VMEM((_KA, _N), jnp.float32),
            pltpu.VMEM((_KA, _N), jnp.float32),
        ],
    )(p_t, g_t)

    return sums[0]
